# baseline (speedup 1.0000x reference)
import functools

import jax
import jax.numpy as jnp
from jax import lax
from jax.experimental import pallas as pl
from jax.experimental.pallas import tpu as pltpu
from jax.experimental.pallas import tpu_sc as plsc

VOCAB = 1000000
BATCH = 16384
EMBED = 64
LANES = 128

_info = plsc.get_sparse_core_info()
_NC, _NS = _info.num_cores, _info.num_subcores
_NW = _NC * _NS
_BPW = BATCH // _NW
_CHUNK = 128
_NCHUNK = _BPW // _CHUNK
_OUT_FLAT_PER_TABLE = _BPW * EMBED

_mesh = plsc.VectorSubcoreMesh(core_axis_name="c", subcore_axis_name="s")


@functools.partial(
    pl.kernel,
    mesh=_mesh,
    out_type=jax.ShapeDtypeStruct((2 * BATCH * EMBED,), jnp.float32),
    scratch_types=[
        pltpu.VMEM((_BPW,), jnp.int32),
        pltpu.VMEM((_NCHUNK, _CHUNK), jnp.int32),
        pltpu.VMEM((_CHUNK, LANES), jnp.float32),
        pltpu.VMEM((_CHUNK, LANES), jnp.float32),
        pltpu.VMEM((_OUT_FLAT_PER_TABLE,), jnp.float32),
        pltpu.SemaphoreType.DMA,
        pltpu.SemaphoreType.DMA,
    ],
)
def _sgns_lookup(words_hbm, contexts_hbm, w_table_hbm, c_table_hbm, out_hbm,
                 idx_v, ridx_v, pair0, pair1, out_v, sem0, sem1):
    wid = lax.axis_index("s") * _NC + lax.axis_index("c")
    base = wid * _BPW
    pairs = (pair0, pair1)
    sems = (sem0, sem1)

    def do_table(idx_hbm, tbl_hbm, out_flat_base):
        pltpu.sync_copy(idx_hbm.at[pl.ds(base, _BPW)], idx_v)
        for j in range(_NCHUNK):
            for k in range(_CHUNK // 16):
                v = idx_v[pl.ds(j * _CHUNK + k * 16, 16)]
                ridx_v[j, pl.ds(k * 16, 16)] = lax.shift_right_logical(v, 1)
        cps = [None] * _NCHUNK
        cps[0] = pltpu.async_copy(tbl_hbm.at[ridx_v.at[0]], pairs[0], sems[0])
        for j in range(_NCHUNK):
            if j + 1 < _NCHUNK:
                cps[j + 1] = pltpu.async_copy(
                    tbl_hbm.at[ridx_v.at[j + 1]], pairs[(j + 1) % 2],
                    sems[(j + 1) % 2])
            cps[j].wait()
            pairbuf = pairs[j % 2]

            def body(g, _, j=j, pairbuf=pairbuf):
                rbase = g * 16
                pvec = (idx_v[pl.ds(j * _CHUNK + rbase, 16)] & 1) * EMBED
                for u in range(16):
                    p = pvec[u]
                    db = (j * _CHUNK + rbase + u) * EMBED
                    for k in range(EMBED // 16):
                        out_v[pl.ds(db + k * 16, 16)] = (
                            pairbuf[rbase + u, pl.ds(p + k * 16, 16)])
                return 0

            lax.fori_loop(0, _CHUNK // 16, body, 0)
        pltpu.sync_copy(out_v, out_hbm.at[pl.ds(out_flat_base, _OUT_FLAT_PER_TABLE)])

    do_table(words_hbm, w_table_hbm, base * EMBED)
    do_table(contexts_hbm, c_table_hbm, (BATCH + base) * EMBED)


def kernel(words, contexts, w_table, c_table):
    w2 = w_table.reshape(VOCAB // 2, 2 * EMBED)
    c2 = c_table.reshape(VOCAB // 2, 2 * EMBED)
    flat = _sgns_lookup(words, contexts, w2, c2)
    return flat.reshape(2, BATCH, EMBED)


# speedup vs baseline: 1.0029x; 1.0029x over previous
import functools

import jax
import jax.numpy as jnp
from jax import lax
from jax.experimental import pallas as pl
from jax.experimental.pallas import tpu as pltpu
from jax.experimental.pallas import tpu_sc as plsc

VOCAB = 1000000
BATCH = 16384
EMBED = 64
LANES = 128

_info = plsc.get_sparse_core_info()
_NC, _NS = _info.num_cores, _info.num_subcores
_NW = _NC * _NS
_BPW = BATCH // _NW
_CHUNK = 128
_NCHUNK = _BPW // _CHUNK
_OUT_FLAT = _BPW * EMBED

_mesh = plsc.VectorSubcoreMesh(core_axis_name="c", subcore_axis_name="s")


@functools.partial(
    pl.kernel,
    mesh=_mesh,
    out_type=jax.ShapeDtypeStruct((BATCH * EMBED,), jnp.float32),
    scratch_types=[
        pltpu.VMEM((_BPW,), jnp.int32),
        pltpu.VMEM((_NCHUNK, _CHUNK), jnp.int32),
        pltpu.VMEM((_CHUNK, LANES), jnp.float32),
        pltpu.VMEM((_CHUNK, LANES), jnp.float32),
        pltpu.VMEM((_OUT_FLAT,), jnp.float32),
        pltpu.SemaphoreType.DMA,
        pltpu.SemaphoreType.DMA,
    ],
)
def _lookup_one(idx_hbm, tbl_hbm, out_hbm,
                idx_v, ridx_v, pair0, pair1, out_v, sem0, sem1):
    wid = lax.axis_index("s") * _NC + lax.axis_index("c")
    base = wid * _BPW
    pairs = (pair0, pair1)
    sems = (sem0, sem1)

    pltpu.sync_copy(idx_hbm.at[pl.ds(base, _BPW)], idx_v)
    for j in range(_NCHUNK):
        for k in range(_CHUNK // 16):
            v = idx_v[pl.ds(j * _CHUNK + k * 16, 16)]
            ridx_v[j, pl.ds(k * 16, 16)] = lax.shift_right_logical(v, 1)
    cps = [None] * _NCHUNK
    cps[0] = pltpu.async_copy(tbl_hbm.at[ridx_v.at[0]], pairs[0], sems[0])
    for j in range(_NCHUNK):
        if j + 1 < _NCHUNK:
            cps[j + 1] = pltpu.async_copy(
                tbl_hbm.at[ridx_v.at[j + 1]], pairs[(j + 1) % 2],
                sems[(j + 1) % 2])
        cps[j].wait()
        pairbuf = pairs[j % 2]

        def body(g, _, j=j, pairbuf=pairbuf):
            rbase = g * 16
            pvec = (idx_v[pl.ds(j * _CHUNK + rbase, 16)] & 1) * EMBED
            for u in range(16):
                p = pvec[u]
                db = (j * _CHUNK + rbase + u) * EMBED
                for k in range(EMBED // 16):
                    out_v[pl.ds(db + k * 16, 16)] = (
                        pairbuf[rbase + u, pl.ds(p + k * 16, 16)])
            return 0

        lax.fori_loop(0, _CHUNK // 16, body, 0)
    pltpu.sync_copy(out_v, out_hbm.at[pl.ds(base * EMBED, _OUT_FLAT)])


def kernel(words, contexts, w_table, c_table):
    w2 = w_table.reshape(VOCAB // 2, 2 * EMBED)
    c2 = c_table.reshape(VOCAB // 2, 2 * EMBED)
    w_out = _lookup_one(words, w2)
    c_out = _lookup_one(contexts, c2)
    return jnp.stack(
        [w_out.reshape(BATCH, EMBED), c_out.reshape(BATCH, EMBED)], axis=0)
